# baseline (speedup 1.0000x reference)
import functools

import jax
import jax.numpy as jnp
import numpy as np
from jax import lax
from jax.experimental import pallas as pl
from jax.experimental.pallas import tpu as pltpu
from jax.experimental.pallas import tpu_sc as plsc


@functools.cache
def _blue_params(n_sample, set_length, nh, nw):
    rng = np.random.default_rng(0)
    idx = [int(v) for v in rng.integers(0, set_length, size=(n_sample,))]
    P = np.zeros((n_sample, nh), np.int32)
    Q = np.zeros((n_sample, nw), np.int32)
    PC = np.zeros((n_sample, nh), np.int32)
    QC = np.zeros((n_sample, nw), np.int32)
    i = np.arange(nh)
    for s in range(n_sample):
        f1 = bool(rng.random() < 0.5)
        f2 = bool(rng.random() < 0.5)
        f3 = bool(rng.random() < 0.5)
        rh = int(rng.integers(0, nh))
        rw = int(rng.integers(0, nw))
        sr = lambda p: (nh - 1 - p) if f2 else p
        sc = lambda q: (nw - 1 - q) if f1 else q
        if not f3:
            P[s] = sr((i - rh) % nh)
            QC[s] = sc((i - rw) % nw)
        else:
            PC[s] = sc(nh - 1 - ((i - rh) % nh))
            Q[s] = sr((i - rw) % nw)
    return idx, P, Q, PC, QC


def _sc_transform(data, B, S, NH, NW, W):
    idx, P, Q, PC, QC = _blue_params(B, S, NH, NW)
    PPC = np.concatenate([P, PC], axis=1)
    QQC = np.concatenate([Q, QC], axis=1)
    L = 16
    NCHUNK = 4
    RPC = NH // 2 // NCHUNK
    f32, i32 = jnp.float32, jnp.int32

    mesh = plsc.VectorSubcoreMesh(core_axis_name="c", subcore_axis_name="s")

    @functools.partial(
        pl.kernel,
        out_type=jax.ShapeDtypeStruct((B * NH, W), f32),
        mesh=mesh,
        compiler_params=pltpu.CompilerParams(needs_layout_passes=False),
        scratch_types=[
            pltpu.VMEM((NH, NW), f32),
            pltpu.VMEM((2 * NH,), i32),
            pltpu.VMEM((2 * NW,), i32),
            pltpu.VMEM((RPC, W), f32),
            pltpu.VMEM((RPC, W), f32),
            pltpu.VMEM((RPC, W), f32),
            pltpu.SemaphoreType.DMA,
            pltpu.SemaphoreType.DMA,
            pltpu.SemaphoreType.DMA,
            pltpu.SemaphoreType.DMA,
        ],
    )
    def run(data_hbm, ppc_hbm, qqc_hbm, out_hbm,
            tile_v, ppc_v, qqc_v, buf0, buf1, buf2,
            sem_st, sem0, sem1, sem2):
        b = lax.axis_index("s")
        half = lax.axis_index("c")

        idxb = jnp.int32(0)
        for s, v in enumerate(idx):
            idxb = idxb + jnp.where(b == s, jnp.int32(v), jnp.int32(0))
        h_st = pltpu.async_copy(
            data_hbm.at[pl.ds(idxb * NH, NH), :], tile_v, sem_st)
        h_p = pltpu.async_copy(
            ppc_hbm.at[pl.ds(b * 2 * NH, 2 * NH)], ppc_v, sem_st)
        h_q = pltpu.async_copy(
            qqc_hbm.at[pl.ds(b * 2 * NW, 2 * NW)], qqc_v, sem_st)
        h_st.wait()
        h_p.wait()
        h_q.wait()

        bufs = (buf0, buf1, buf2)
        sems = (sem0, sem1, sem2)
        pending = [None, None, None]
        for t in range(NCHUNK):
            slot = t % 3
            if pending[slot] is not None:
                pending[slot].wait()
            buf = bufs[slot]
            base = half * (NH // 2) + t * RPC

            def row_body(ii, _, buf=buf, base=base):
                i = base + ii
                pb = plsc.load_gather(ppc_v, [jnp.full((L,), i, i32)])
                pcb = plsc.load_gather(ppc_v, [jnp.full((L,), NH + i, i32)])
                for k in range(NW // L):
                    rv = pb + qqc_v[pl.ds(k * L, L)]
                    cv = pcb + qqc_v[pl.ds(NW + k * L, L)]
                    vals = plsc.load_gather(tile_v, [rv, cv])
                    buf[ii, pl.ds(k * L, L)] = vals
                    buf[ii, pl.ds(NW + k * L, L)] = vals
                return 0

            lax.fori_loop(0, RPC, row_body, 0)
            pending[slot] = pltpu.async_copy(
                buf, out_hbm.at[pl.ds(b * NH + base, RPC), :], sems[slot])

        for slot in range(3):
            if pending[slot] is not None:
                pending[slot].wait()

    return run(data.reshape(S * NH, NW),
               jnp.asarray(PPC).reshape(-1),
               jnp.asarray(QQC).reshape(-1))


def kernel(x, data):
    B, C, H, W = x.shape
    S, NH, NW = data.shape
    assert H == 2 * NH and W == 2 * NW
    f32 = jnp.float32

    tiles = _sc_transform(data, B, S, NH, NW, W)

    def body(d_ref, o_ref, rows0, rows1, sem0, sem1):
        b = pl.program_id(0)
        scratch = [(rows0, sem0), (rows1, sem1)]

        def wait_slot(par):
            rows, sem = scratch[par]
            for c in range(C):
                for v in range(2):
                    pltpu.make_async_copy(
                        rows, o_ref.at[0, c, pl.ds(v * NH, NH), :], sem
                    ).wait()

        def fill_and_fire(par, bb):
            rows, sem = scratch[par]
            rows[...] = d_ref[0]
            for c in range(C):
                for v in range(2):
                    pltpu.make_async_copy(
                        rows, o_ref.at[bb, c, pl.ds(v * NH, NH), :], sem
                    ).start()

        for par in range(2):
            @pl.when((b >= 2) & (b % 2 == par))
            def _():
                wait_slot(par)

        for par in range(2):
            @pl.when(b % 2 == par)
            def _():
                fill_and_fire(par, b)

        @pl.when(b == B - 1)
        def _():
            for par in range(2):
                wait_slot(par)

    return pl.pallas_call(
        body,
        grid=(B,),
        in_specs=[pl.BlockSpec((1, NH, W), lambda b: (b, 0, 0))],
        out_specs=pl.BlockSpec(memory_space=pltpu.HBM),
        out_shape=jax.ShapeDtypeStruct((B, C, H, W), f32),
        scratch_shapes=[
            pltpu.VMEM((NH, W), f32),
            pltpu.VMEM((NH, W), f32),
            pltpu.SemaphoreType.DMA,
            pltpu.SemaphoreType.DMA,
        ],
    )(tiles.reshape(B, NH, W))


# speedup vs baseline: 1.6012x; 1.6012x over previous
import functools

import jax
import jax.numpy as jnp
import numpy as np
from jax import lax
from jax.experimental import pallas as pl
from jax.experimental.pallas import tpu as pltpu
from jax.experimental.pallas import tpu_sc as plsc


@functools.cache
def _blue_params(n_sample, set_length, nh, nw):
    rng = np.random.default_rng(0)
    idx = [int(v) for v in rng.integers(0, set_length, size=(n_sample,))]
    P = np.zeros((n_sample, nh), np.int32)
    Q = np.zeros((n_sample, nw), np.int32)
    PC = np.zeros((n_sample, nh), np.int32)
    QC = np.zeros((n_sample, nw), np.int32)
    i = np.arange(nh)
    for s in range(n_sample):
        f1 = bool(rng.random() < 0.5)
        f2 = bool(rng.random() < 0.5)
        f3 = bool(rng.random() < 0.5)
        rh = int(rng.integers(0, nh))
        rw = int(rng.integers(0, nw))
        sr = lambda p: (nh - 1 - p) if f2 else p
        sc = lambda q: (nw - 1 - q) if f1 else q
        if not f3:
            P[s] = sr((i - rh) % nh)
            QC[s] = sc((i - rw) % nw)
        else:
            PC[s] = sc(nh - 1 - ((i - rh) % nh))
            Q[s] = sr((i - rw) % nw)
    return idx, P, Q, PC, QC


def kernel(x, data):
    B, C, H, W = x.shape
    S, NH, NW = data.shape
    assert H == 2 * NH and W == 2 * NW
    idx, P, Q, PC, QC = _blue_params(B, S, NH, NW)
    PPC = np.concatenate([P, PC], axis=1)
    QQC = np.concatenate([Q, QC], axis=1)
    L = 16
    NCHUNK = 4
    RPC = NH // 2 // NCHUNK
    f32, i32 = jnp.float32, jnp.int32

    mesh = plsc.VectorSubcoreMesh(core_axis_name="c", subcore_axis_name="s")

    @functools.partial(
        pl.kernel,
        out_type=jax.ShapeDtypeStruct((B * C * H, W), f32),
        mesh=mesh,
        compiler_params=pltpu.CompilerParams(needs_layout_passes=False),
        scratch_types=[
            pltpu.VMEM((NH, NW), f32),
            pltpu.VMEM((2 * NH,), i32),
            pltpu.VMEM((2 * NW,), i32),
            pltpu.VMEM((RPC, W), f32),
            pltpu.VMEM((RPC, W), f32),
            pltpu.VMEM((RPC, W), f32),
            pltpu.SemaphoreType.DMA,
            pltpu.SemaphoreType.DMA,
            pltpu.SemaphoreType.DMA,
            pltpu.SemaphoreType.DMA,
        ],
    )
    def run(data_hbm, ppc_hbm, qqc_hbm, out_hbm,
            tile_v, ppc_v, qqc_v, buf0, buf1, buf2,
            sem_st, sem0, sem1, sem2):
        b = lax.axis_index("s")
        half = lax.axis_index("c")

        idxb = jnp.int32(0)
        for s, v in enumerate(idx):
            idxb = idxb + jnp.where(b == s, jnp.int32(v), jnp.int32(0))
        h_st = pltpu.async_copy(
            data_hbm.at[pl.ds(idxb * NH, NH), :], tile_v, sem_st)
        h_p = pltpu.async_copy(
            ppc_hbm.at[pl.ds(b * 2 * NH, 2 * NH)], ppc_v, sem_st)
        h_q = pltpu.async_copy(
            qqc_hbm.at[pl.ds(b * 2 * NW, 2 * NW)], qqc_v, sem_st)
        h_st.wait()
        h_p.wait()
        h_q.wait()

        qs = [qqc_v[pl.ds(k * L, L)] for k in range(NW // L)]
        qcs = [qqc_v[pl.ds(NW + k * L, L)] for k in range(NW // L)]

        bufs = (buf0, buf1, buf2)
        sems = (sem0, sem1, sem2)
        pending = [None, None, None]
        UR = 2
        for t in range(NCHUNK):
            slot = t % 3
            if pending[slot] is not None:
                for h in pending[slot]:
                    h.wait()
            buf = bufs[slot]
            base = half * (NH // 2) + t * RPC

            def row_body(jj, _, buf=buf, base=base):
                for u in range(UR):
                    ii = jj * UR + u
                    i = base + ii
                    iv = jnp.full((L,), i, i32)
                    pb = plsc.load_gather(ppc_v, [iv])
                    pcb = plsc.load_gather(ppc_v, [iv + NH])
                    for k in range(NW // L):
                        vals = plsc.load_gather(
                            tile_v, [pb + qs[k], pcb + qcs[k]])
                        buf[ii, pl.ds(k * L, L)] = vals
                        buf[ii, pl.ds(NW + k * L, L)] = vals
                return 0

            lax.fori_loop(0, RPC // UR, row_body, 0)

            hs = []
            for c in range(C):
                for v in range(2):
                    row0 = (b * C + c) * H + v * NH + base
                    hs.append(pltpu.async_copy(
                        buf, out_hbm.at[pl.ds(row0, RPC), :], sems[slot]))
            pending[slot] = hs

        for slot in range(3):
            if pending[slot] is not None:
                for h in pending[slot]:
                    h.wait()

    out = run(data.reshape(S * NH, NW),
              jnp.asarray(PPC).reshape(-1),
              jnp.asarray(QQC).reshape(-1))
    return out.reshape(B, C, H, W)
